# baseline (speedup 1.0000x reference)
import functools

import jax
import jax.numpy as jnp
from jax import lax
from jax.experimental import pallas as pl
from jax.experimental.pallas import tpu as pltpu
from jax.experimental.pallas import tpu_sc as plsc

N_NODES_K = 10000
N_EDGES_K = 320000
CH = 128
NREL = 24

NC = 2
NS = 16
NW = NC * NS
CHUNK = 80
EDGES_PER_W = N_EDGES_K // NW
CHUNKS_PER_W = EDGES_PER_W // CHUNK
NSLOT = 4
NGRP = CHUNKS_PER_W // NSLOT
NTAIL = CHUNKS_PER_W - NGRP * NSLOT
ROWS_PER_SUB = 624
ROWS_TAIL = N_NODES_K - NS * ROWS_PER_SUB


def _expand_table(all_emb, weight):
    def body(a_ref, w_ref, o_ref):
        a = a_ref[...]
        w = w_ref[...]
        o_ref[...] = w[:, None, :] * a[None, :, :]

    t = pl.pallas_call(
        body,
        out_shape=jax.ShapeDtypeStruct((NREL, N_NODES_K, CH), jnp.float32),
        grid=(5,),
        in_specs=[
            pl.BlockSpec((N_NODES_K // 5, CH), lambda i: (i, 0)),
            pl.BlockSpec((NREL, CH), lambda i: (0, 0)),
        ],
        out_specs=pl.BlockSpec((NREL, N_NODES_K // 5, CH), lambda i: (0, i, 0)),
    )(all_emb, weight)
    return t.reshape(NREL * N_NODES_K, CH)


def _sc_aggregate(table, edge_flat, etype):
    mesh = plsc.VectorSubcoreMesh(core_axis_name="c", subcore_axis_name="s")

    @functools.partial(
        pl.kernel,
        mesh=mesh,
        out_type=jax.ShapeDtypeStruct((NC, N_NODES_K, CH), jnp.float32),
        scratch_types=(
            [pltpu.VMEM((CHUNK,), jnp.int32) for _ in range(NSLOT)]
            + [pltpu.VMEM((CHUNK,), jnp.int32) for _ in range(NSLOT)]
            + [pltpu.VMEM((CHUNK,), jnp.int32) for _ in range(NSLOT)]
            + [pltpu.VMEM((CHUNK,), jnp.int32) for _ in range(NSLOT)]
            + [pltpu.VMEM((CHUNK, CH), jnp.float32) for _ in range(NSLOT)]
            + [pltpu.VMEM_SHARED((N_NODES_K, CH), jnp.float32)]
            + [pltpu.SemaphoreType.DMA for _ in range(3 * NSLOT)]
        ),
    )
    def k(table_hbm, edge_hbm, etype_hbm, out_hbm, *scratch):
        comb_b = scratch[0:NSLOT]
        head_b = scratch[NSLOT:2 * NSLOT]
        tail_b = scratch[2 * NSLOT:3 * NSLOT]
        etv_b = scratch[3 * NSLOT:4 * NSLOT]
        rows_b = scratch[4 * NSLOT:5 * NSLOT]
        acc = scratch[5 * NSLOT]
        isem = scratch[5 * NSLOT + 1:5 * NSLOT + 1 + NSLOT]
        gsem = scratch[5 * NSLOT + 1 + NSLOT:5 * NSLOT + 1 + 2 * NSLOT]
        ssem = scratch[5 * NSLOT + 1 + 2 * NSLOT:5 * NSLOT + 1 + 3 * NSLOT]

        cid = lax.axis_index("c")
        sid = lax.axis_index("s")
        wid = cid * NS + sid

        def idx_copies(j, s):
            base = wid * EDGES_PER_W + j * CHUNK
            return (
                pltpu.make_async_copy(edge_hbm.at[pl.ds(base, CHUNK)],
                                      head_b[s], isem[s]),
                pltpu.make_async_copy(edge_hbm.at[pl.ds(N_EDGES_K + base, CHUNK)],
                                      tail_b[s], isem[s]),
                pltpu.make_async_copy(etype_hbm.at[pl.ds(base, CHUNK)],
                                      etv_b[s], isem[s]),
            )

        for s in range(NSLOT):
            for c in idx_copies(s, s):
                c.start()

        def zbody(e, _):
            for s in range(CH // 16):
                rows_b[0][e, pl.ds(s * 16, 16)] = jnp.zeros((16,), jnp.float32)
            return 0
        lax.fori_loop(0, CHUNK, zbody, 0)

        arow = sid * ROWS_PER_SUB
        for i in range(ROWS_PER_SUB // CHUNK):
            pltpu.sync_copy(rows_b[0], acc.at[pl.ds(arow + i * CHUNK, CHUNK)])
        rem = ROWS_PER_SUB - (ROWS_PER_SUB // CHUNK) * CHUNK
        pltpu.sync_copy(rows_b[0].at[pl.ds(0, rem)],
                        acc.at[pl.ds(arow + (ROWS_PER_SUB // CHUNK) * CHUNK, rem)])

        @pl.when(sid == NS - 1)
        def _zero_tail():
            pltpu.sync_copy(rows_b[0].at[pl.ds(0, ROWS_TAIL)],
                            acc.at[pl.ds(NS * ROWS_PER_SUB, ROWS_TAIL)])

        plsc.subcore_barrier()

        def make_comb(s):
            for q in range(CHUNK // 16):
                sl = pl.ds(q * 16, 16)
                comb_b[s][sl] = etv_b[s][sl] * N_NODES_K + tail_b[s][sl]

        def gather_copy(s):
            return pltpu.make_async_copy(table_hbm.at[comb_b[s]], rows_b[s],
                                         gsem[s])

        def scatter_start(s):
            pltpu.async_copy(rows_b[s], acc.at[head_b[s]], ssem[s], add=True)

        def scatter_wait(s):
            pltpu.make_async_copy(rows_b[s], acc.at[head_b[s]], ssem[s]).wait()

        for s in range(NSLOT):
            for c in idx_copies(s, s):
                c.wait()
            make_comb(s)
            gather_copy(s).start()
        for s in range(NSLOT):
            gather_copy(s).wait()
            scatter_start(s)

        def grp_body(t, _):
            j0 = t * NSLOT
            for s in range(NSLOT):
                scatter_wait(s)
                for c in idx_copies(j0 + s, s):
                    c.start()
            for s in range(NSLOT):
                for c in idx_copies(j0 + s, s):
                    c.wait()
                make_comb(s)
                gather_copy(s).start()
            for s in range(NSLOT):
                gather_copy(s).wait()
                scatter_start(s)
            return 0

        lax.fori_loop(1, NGRP, grp_body, 0)

        for s in range(NTAIL):
            jt = NGRP * NSLOT + s
            scatter_wait(s)
            for c in idx_copies(jt, s):
                c.start()
            for c in idx_copies(jt, s):
                c.wait()
            make_comb(s)
            gather_copy(s).start()
            gather_copy(s).wait()
            scatter_start(s)

        for s in range(NSLOT):
            scatter_wait(s)

        plsc.subcore_barrier()

        pltpu.sync_copy(acc.at[pl.ds(arow, ROWS_PER_SUB)],
                        out_hbm.at[cid, pl.ds(arow, ROWS_PER_SUB)])

        @pl.when(sid == NS - 1)
        def _write_tail():
            pltpu.sync_copy(acc.at[pl.ds(NS * ROWS_PER_SUB, ROWS_TAIL)],
                            out_hbm.at[cid, pl.ds(NS * ROWS_PER_SUB, ROWS_TAIL)])

    return k(table, edge_flat, etype)


def _combine(parts):
    def body(a_ref, o_ref):
        o_ref[...] = a_ref[0] + a_ref[1]

    return pl.pallas_call(
        body,
        out_shape=jax.ShapeDtypeStruct((N_NODES_K, CH), jnp.float32),
        grid=(10,),
        in_specs=[pl.BlockSpec((2, N_NODES_K // 10, CH), lambda i: (0, i, 0))],
        out_specs=pl.BlockSpec((N_NODES_K // 10, CH), lambda i: (i, 0)),
    )(parts)


def kernel(all_emb, edge_index, edge_type, weight):
    edge_flat = edge_index.reshape(2 * N_EDGES_K)
    table = _expand_table(all_emb, weight)
    parts = _sc_aggregate(table, edge_flat, edge_type)
    return _combine(parts)


# speedup vs baseline: 1.0139x; 1.0139x over previous
import functools

import jax
import jax.numpy as jnp
from jax import lax
from jax.experimental import pallas as pl
from jax.experimental.pallas import tpu as pltpu
from jax.experimental.pallas import tpu_sc as plsc

N_NODES_K = 10000
N_EDGES_K = 320000
CH = 128
NREL = 24

NC = 2
NS = 16
NW = NC * NS
CHUNK = 80
EDGES_PER_W = N_EDGES_K // NW
CHUNKS_PER_W = EDGES_PER_W // CHUNK
NSLOT = 4
NGRP = CHUNKS_PER_W // NSLOT
NTAIL = CHUNKS_PER_W - NGRP * NSLOT
ROWS_PER_SUB = 624
ROWS_TAIL = N_NODES_K - NS * ROWS_PER_SUB


def _expand_table(all_emb, weight):
    def body(a_ref, w_ref, o_ref):
        a = a_ref[...]
        w = w_ref[...]
        o_ref[...] = w[:, None, :] * a[None, :, :]

    t = pl.pallas_call(
        body,
        out_shape=jax.ShapeDtypeStruct((NREL, N_NODES_K, CH), jnp.float32),
        grid=(10,),
        in_specs=[
            pl.BlockSpec((N_NODES_K // 10, CH), lambda i: (i, 0)),
            pl.BlockSpec((NREL, CH), lambda i: (0, 0)),
        ],
        out_specs=pl.BlockSpec((NREL, N_NODES_K // 10, CH), lambda i: (0, i, 0)),
    )(all_emb, weight)
    return t.reshape(NREL * N_NODES_K, CH)


def _sc_aggregate(table, edge_flat, etype):
    mesh = plsc.VectorSubcoreMesh(core_axis_name="c", subcore_axis_name="s")

    @functools.partial(
        pl.kernel,
        mesh=mesh,
        out_type=jax.ShapeDtypeStruct((NC, N_NODES_K, CH), jnp.float32),
        scratch_types=(
            [pltpu.VMEM((CHUNK,), jnp.int32) for _ in range(NSLOT)]
            + [pltpu.VMEM((CHUNK,), jnp.int32) for _ in range(NSLOT)]
            + [pltpu.VMEM((CHUNK,), jnp.int32) for _ in range(NSLOT)]
            + [pltpu.VMEM((CHUNK,), jnp.int32) for _ in range(NSLOT)]
            + [pltpu.VMEM((CHUNK, CH), jnp.float32) for _ in range(NSLOT)]
            + [pltpu.VMEM_SHARED((N_NODES_K, CH), jnp.float32)]
            + [pltpu.SemaphoreType.DMA for _ in range(3 * NSLOT)]
        ),
    )
    def k(table_hbm, edge_hbm, etype_hbm, out_hbm, *scratch):
        comb_b = scratch[0:NSLOT]
        head_b = scratch[NSLOT:2 * NSLOT]
        tail_b = scratch[2 * NSLOT:3 * NSLOT]
        etv_b = scratch[3 * NSLOT:4 * NSLOT]
        rows_b = scratch[4 * NSLOT:5 * NSLOT]
        acc = scratch[5 * NSLOT]
        isem = scratch[5 * NSLOT + 1:5 * NSLOT + 1 + NSLOT]
        gsem = scratch[5 * NSLOT + 1 + NSLOT:5 * NSLOT + 1 + 2 * NSLOT]
        ssem = scratch[5 * NSLOT + 1 + 2 * NSLOT:5 * NSLOT + 1 + 3 * NSLOT]

        cid = lax.axis_index("c")
        sid = lax.axis_index("s")
        wid = cid * NS + sid

        def idx_copies(j, s):
            base = wid * EDGES_PER_W + j * CHUNK
            return (
                pltpu.make_async_copy(edge_hbm.at[pl.ds(base, CHUNK)],
                                      head_b[s], isem[s]),
                pltpu.make_async_copy(edge_hbm.at[pl.ds(N_EDGES_K + base, CHUNK)],
                                      tail_b[s], isem[s]),
                pltpu.make_async_copy(etype_hbm.at[pl.ds(base, CHUNK)],
                                      etv_b[s], isem[s]),
            )

        for s in range(NSLOT):
            for c in idx_copies(s, s):
                c.start()

        def zbody(e, _):
            for s in range(CH // 16):
                rows_b[0][e, pl.ds(s * 16, 16)] = jnp.zeros((16,), jnp.float32)
            return 0
        lax.fori_loop(0, CHUNK, zbody, 0)

        arow = sid * ROWS_PER_SUB
        for i in range(ROWS_PER_SUB // CHUNK):
            pltpu.sync_copy(rows_b[0], acc.at[pl.ds(arow + i * CHUNK, CHUNK)])
        rem = ROWS_PER_SUB - (ROWS_PER_SUB // CHUNK) * CHUNK
        pltpu.sync_copy(rows_b[0].at[pl.ds(0, rem)],
                        acc.at[pl.ds(arow + (ROWS_PER_SUB // CHUNK) * CHUNK, rem)])

        @pl.when(sid == NS - 1)
        def _zero_tail():
            pltpu.sync_copy(rows_b[0].at[pl.ds(0, ROWS_TAIL)],
                            acc.at[pl.ds(NS * ROWS_PER_SUB, ROWS_TAIL)])

        plsc.subcore_barrier()

        def make_comb(s):
            for q in range(CHUNK // 16):
                sl = pl.ds(q * 16, 16)
                comb_b[s][sl] = etv_b[s][sl] * N_NODES_K + tail_b[s][sl]

        def gather_copy(s):
            return pltpu.make_async_copy(table_hbm.at[comb_b[s]], rows_b[s],
                                         gsem[s])

        def scatter_start(s):
            pltpu.async_copy(rows_b[s], acc.at[head_b[s]], ssem[s], add=True)

        def scatter_wait(s):
            pltpu.make_async_copy(rows_b[s], acc.at[head_b[s]], ssem[s]).wait()

        for s in range(NSLOT):
            for c in idx_copies(s, s):
                c.wait()
            make_comb(s)
            gather_copy(s).start()
        for s in range(NSLOT):
            gather_copy(s).wait()
            scatter_start(s)

        def grp_body(t, _):
            j0 = t * NSLOT
            for s in range(NSLOT):
                scatter_wait(s)
                for c in idx_copies(j0 + s, s):
                    c.start()
            for s in range(NSLOT):
                for c in idx_copies(j0 + s, s):
                    c.wait()
                make_comb(s)
                gather_copy(s).start()
            for s in range(NSLOT):
                gather_copy(s).wait()
                scatter_start(s)
            return 0

        lax.fori_loop(1, NGRP, grp_body, 0)

        for s in range(NTAIL):
            jt = NGRP * NSLOT + s
            scatter_wait(s)
            for c in idx_copies(jt, s):
                c.start()
            for c in idx_copies(jt, s):
                c.wait()
            make_comb(s)
            gather_copy(s).start()
            gather_copy(s).wait()
            scatter_start(s)

        for s in range(NSLOT):
            scatter_wait(s)

        plsc.subcore_barrier()

        pltpu.sync_copy(acc.at[pl.ds(arow, ROWS_PER_SUB)],
                        out_hbm.at[cid, pl.ds(arow, ROWS_PER_SUB)])

        @pl.when(sid == NS - 1)
        def _write_tail():
            pltpu.sync_copy(acc.at[pl.ds(NS * ROWS_PER_SUB, ROWS_TAIL)],
                            out_hbm.at[cid, pl.ds(NS * ROWS_PER_SUB, ROWS_TAIL)])

    return k(table, edge_flat, etype)


def _combine(parts):
    def body(a_ref, o_ref):
        o_ref[...] = a_ref[0] + a_ref[1]

    return pl.pallas_call(
        body,
        out_shape=jax.ShapeDtypeStruct((N_NODES_K, CH), jnp.float32),
        grid=(10,),
        in_specs=[pl.BlockSpec((2, N_NODES_K // 10, CH), lambda i: (0, i, 0))],
        out_specs=pl.BlockSpec((N_NODES_K // 10, CH), lambda i: (i, 0)),
    )(parts)


def kernel(all_emb, edge_index, edge_type, weight):
    edge_flat = edge_index.reshape(2 * N_EDGES_K)
    table = _expand_table(all_emb, weight)
    parts = _sc_aggregate(table, edge_flat, edge_type)
    return _combine(parts)
